# Initial kernel scaffold; baseline (speedup 1.0000x reference)
import jax
import jax.numpy as jnp
from jax.experimental import pallas as pl


def kernel(idx, embed):
    raise NotImplementedError("write your pallas kernel here")


# speedup vs baseline: 1.0145x; 1.0145x over previous
import functools

import jax
import jax.numpy as jnp
from jax import lax
from jax.experimental import pallas as pl
from jax.experimental.pallas import tpu as pltpu
from jax.experimental.pallas import tpu_sc as plsc

_NC = 2
_NS = 16
_NW = _NC * _NS
_R = 64


def _make_gather(nchunks: int, rows: int, vocab: int, d: int):
    mesh = plsc.VectorSubcoreMesh(core_axis_name="c", subcore_axis_name="s")

    @functools.partial(
        pl.kernel,
        mesh=mesh,
        out_type=jax.ShapeDtypeStruct((_NW, nchunks, rows, d), jnp.float32),
        scratch_types=[
            pltpu.VMEM((nchunks, rows), jnp.int32),
            pltpu.VMEM((rows, d), jnp.float32),
            pltpu.SemaphoreType.DMA,
        ],
        compiler_params=pltpu.CompilerParams(use_tc_tiling_on_sc=False),
    )
    def body(idx_hbm, table_hbm, out_hbm, idx_v, buf_v, sem):
        wid = lax.axis_index("s") * _NC + lax.axis_index("c")
        pltpu.sync_copy(idx_hbm.at[wid], idx_v)

        def step(c, carry):
            pltpu.async_copy(table_hbm.at[idx_v.at[c]], buf_v, sem).wait()
            pltpu.sync_copy(buf_v, out_hbm.at[wid, c])
            return carry

        lax.fori_loop(0, nchunks, step, 0)

    return body


def kernel(idx, embed):
    b, l = idx.shape
    vocab, d = embed.shape
    total = b * l
    assert total % (_NW * _R) == 0
    nchunks = total // (_NW * _R)
    idx_r = idx.reshape(_NW, nchunks, _R).astype(jnp.int32)
    out = _make_gather(nchunks, _R, vocab, d)(idx_r, embed)
    return out.reshape(b, l, d)


# speedup vs baseline: 1.0337x; 1.0190x over previous
import functools

import jax
import jax.numpy as jnp
from jax import lax
from jax.experimental import pallas as pl
from jax.experimental.pallas import tpu as pltpu
from jax.experimental.pallas import tpu_sc as plsc

_NC = 2
_NS = 16
_NW = _NC * _NS
_R = 16
_NBUF = 4


def _make_gather(nchunks: int, rows: int, vocab: int, d: int):
    mesh = plsc.VectorSubcoreMesh(core_axis_name="c", subcore_axis_name="s")
    assert nchunks % _NBUF == 0

    @functools.partial(
        pl.kernel,
        mesh=mesh,
        out_type=jax.ShapeDtypeStruct((_NW, nchunks, rows, d), jnp.float32),
        scratch_types=[
            pltpu.VMEM((nchunks, rows), jnp.int32),
        ]
        + [pltpu.VMEM((rows, d), jnp.float32) for _ in range(_NBUF)]
        + [pltpu.SemaphoreType.DMA for _ in range(2 * _NBUF)],
        compiler_params=pltpu.CompilerParams(use_tc_tiling_on_sc=False),
    )
    def body(idx_hbm, table_hbm, out_hbm, idx_v, *bufs_and_sems):
        bufs = bufs_and_sems[:_NBUF]
        gsem = bufs_and_sems[_NBUF:2 * _NBUF]
        wsem = bufs_and_sems[2 * _NBUF:]
        wid = lax.axis_index("s") * _NC + lax.axis_index("c")
        pltpu.sync_copy(idx_hbm.at[wid], idx_v)

        pltpu.async_copy(table_hbm.at[idx_v.at[0]], bufs[0], gsem[0])
        pltpu.async_copy(table_hbm.at[idx_v.at[1]], bufs[1], gsem[1])

        def quad(i, carry):
            for k in range(_NBUF):
                c = _NBUF * i + k
                nb = (k + 2) % _NBUF

                @pl.when(c >= 2)
                def _():
                    pltpu.make_async_copy(
                        bufs[nb], out_hbm.at[wid, c - 2], wsem[nb]).wait()

                @pl.when(c + 2 < nchunks)
                def _():
                    pltpu.async_copy(
                        table_hbm.at[idx_v.at[c + 2]], bufs[nb], gsem[nb])

                pltpu.make_async_copy(
                    table_hbm.at[idx_v.at[c]], bufs[k], gsem[k]).wait()
                pltpu.async_copy(bufs[k], out_hbm.at[wid, c], wsem[k])
            return carry

        lax.fori_loop(0, nchunks // _NBUF, quad, 0)

        for c in (nchunks - 2, nchunks - 1):
            b = c % _NBUF
            pltpu.make_async_copy(bufs[b], out_hbm.at[wid, c], wsem[b]).wait()

    return body


def kernel(idx, embed):
    b, l = idx.shape
    vocab, d = embed.shape
    total = b * l
    assert total % (_NW * _R) == 0
    nchunks = total // (_NW * _R)
    idx_r = idx.reshape(_NW, nchunks, _R).astype(jnp.int32)
    out = _make_gather(nchunks, _R, vocab, d)(idx_r, embed)
    return out.reshape(b, l, d)


# speedup vs baseline: 1.0364x; 1.0026x over previous
import functools

import jax
import jax.numpy as jnp
from jax import lax
from jax.experimental import pallas as pl
from jax.experimental.pallas import tpu as pltpu
from jax.experimental.pallas import tpu_sc as plsc

_NC = 2
_NS = 16
_NW = _NC * _NS


def _make_gather(b: int, l: int, vocab: int, d: int):
    mesh = plsc.VectorSubcoreMesh(core_axis_name="c", subcore_axis_name="s")
    bpw = b // _NW

    @functools.partial(
        pl.kernel,
        mesh=mesh,
        out_type=jax.ShapeDtypeStruct((b, l, d), jnp.float32),
        scratch_types=[
            pltpu.VMEM((bpw, l), jnp.int32),
            pltpu.VMEM((l, d), jnp.float32),
            pltpu.VMEM((l, d), jnp.float32),
            pltpu.SemaphoreType.DMA,
            pltpu.SemaphoreType.DMA,
            pltpu.SemaphoreType.DMA,
            pltpu.SemaphoreType.DMA,
        ],
        compiler_params=pltpu.CompilerParams(use_tc_tiling_on_sc=False),
    )
    def body(idx_hbm, table_hbm, out_hbm, idx_v, buf0, buf1, g0, g1, w0, w1):
        bufs = (buf0, buf1)
        gsem = (g0, g1)
        wsem = (w0, w1)
        wid = lax.axis_index("s") * _NC + lax.axis_index("c")
        base = wid * bpw
        pltpu.sync_copy(idx_hbm.at[wid], idx_v)

        pltpu.async_copy(table_hbm.at[idx_v.at[0]], bufs[0], gsem[0])

        def pair(i, carry):
            for k in range(2):
                c = 2 * i + k
                nb = (k + 1) % 2

                @pl.when(c >= 1)
                def _():
                    pltpu.make_async_copy(
                        bufs[nb], out_hbm.at[base + c - 1], wsem[nb]).wait()

                @pl.when(c + 1 < bpw)
                def _():
                    pltpu.async_copy(
                        table_hbm.at[idx_v.at[c + 1]], bufs[nb], gsem[nb])

                pltpu.make_async_copy(
                    table_hbm.at[idx_v.at[c]], bufs[k], gsem[k]).wait()
                pltpu.async_copy(bufs[k], out_hbm.at[base + c], wsem[k])
            return carry

        lax.fori_loop(0, bpw // 2, pair, 0)

        c = bpw - 1
        pltpu.make_async_copy(
            bufs[c % 2], out_hbm.at[base + c], wsem[c % 2]).wait()

    return body


def kernel(idx, embed):
    b, l = idx.shape
    vocab, d = embed.shape
    assert b % (2 * _NW) == 0
    idx_r = idx.reshape(_NW, b // _NW, l).astype(jnp.int32)
    return _make_gather(b, l, vocab, d)(idx_r, embed)


# speedup vs baseline: 1.5393x; 1.4852x over previous
import functools

import jax
import jax.numpy as jnp
from jax import lax
from jax.experimental import pallas as pl
from jax.experimental.pallas import tpu as pltpu
from jax.experimental.pallas import tpu_sc as plsc

_NC = 2
_NS = 16
_NW = _NC * _NS
_L16 = 16


def _make_gather(b: int, l: int, vocab: int, d: int):
    mesh = plsc.VectorSubcoreMesh(core_axis_name="c", subcore_axis_name="s")
    bpw = b // _NW
    nfull = d // 128
    tail = d - nfull * 128

    @functools.partial(
        pl.kernel,
        mesh=mesh,
        out_type=jax.ShapeDtypeStruct((b, l, d), jnp.float32),
        scratch_types=[
            pltpu.VMEM((bpw, l), jnp.int32),
            pltpu.VMEM((l, d), jnp.float32),
            pltpu.VMEM((l, 128), jnp.float32),
            pltpu.SemaphoreType.DMA,
            pltpu.SemaphoreType.DMA,
        ],
    )
    def body(idx_hbm, table_hbm, tailt_hbm, out_hbm, idx_v, buf, tbuf, g0, w0):
        wid = lax.axis_index("s") * _NC + lax.axis_index("c")
        base = wid * bpw
        pltpu.sync_copy(idx_hbm.at[wid], idx_v)

        def step(c, carry):
            for j in range(nfull):
                pltpu.async_copy(
                    table_hbm.at[idx_v.at[c], pl.ds(j * 128, 128)],
                    buf.at[:, pl.ds(j * 128, 128)], g0)
            pltpu.async_copy(tailt_hbm.at[idx_v.at[c]], tbuf, g0)
            for j in range(nfull):
                pltpu.make_async_copy(
                    table_hbm.at[idx_v.at[c], pl.ds(j * 128, 128)],
                    buf.at[:, pl.ds(j * 128, 128)], g0).wait()
            pltpu.make_async_copy(tailt_hbm.at[idx_v.at[c]], tbuf, g0).wait()

            def blit(r, carry2):
                for k in range(tail // _L16):
                    buf[r, pl.ds(nfull * 128 + k * _L16, _L16)] = (
                        tbuf[r, pl.ds(128 - tail + k * _L16, _L16)])
                if tail % _L16:
                    buf[r, pl.ds(d - _L16, _L16)] = (
                        tbuf[r, pl.ds(128 - _L16, _L16)])
                return carry2

            lax.fori_loop(0, l, blit, 0)

            pltpu.async_copy(buf, out_hbm.at[base + c], w0)
            pltpu.make_async_copy(buf, out_hbm.at[base + c], w0).wait()
            return carry

        lax.fori_loop(0, bpw, step, 0)

    return body


def kernel(idx, embed):
    b, l = idx.shape
    vocab, d = embed.shape
    assert b % (2 * _NW) == 0 and d > 128
    idx_r = idx.reshape(_NW, b // _NW, l).astype(jnp.int32)
    tail_table = embed[:, d - 128:]
    return _make_gather(b, l, vocab, d)(idx_r, embed, tail_table)


# speedup vs baseline: 1.5445x; 1.0034x over previous
import functools

import jax
import jax.numpy as jnp
from jax import lax
from jax.experimental import pallas as pl
from jax.experimental.pallas import tpu as pltpu
from jax.experimental.pallas import tpu_sc as plsc

_NC = 2
_NS = 16
_NW = _NC * _NS
_L16 = 16


def _make_gather(b: int, l: int, vocab: int, d: int):
    mesh = plsc.VectorSubcoreMesh(core_axis_name="c", subcore_axis_name="s")
    bpw = b // _NW
    nfull = d // 128
    tail = d - nfull * 128

    @functools.partial(
        pl.kernel,
        mesh=mesh,
        out_type=jax.ShapeDtypeStruct((b, l, d), jnp.float32),
        scratch_types=[
            pltpu.VMEM((bpw, l), jnp.int32),
            pltpu.VMEM((l, d), jnp.float32),
            pltpu.VMEM((l, 128), jnp.float32),
            pltpu.SemaphoreType.DMA,
            pltpu.SemaphoreType.DMA,
        ],
        compiler_params=pltpu.CompilerParams(needs_layout_passes=False),
    )
    def body(idx_hbm, table_hbm, tailt_hbm, out_hbm, idx_v, buf, tbuf, g0, w0):
        wid = lax.axis_index("s") * _NC + lax.axis_index("c")
        base = wid * bpw
        pltpu.sync_copy(idx_hbm.at[wid], idx_v)

        lanes = lax.iota(jnp.int32, _L16)
        rem_src = jnp.where(lanes < 8, 128 - 8 + lanes, 128 - _L16 + lanes)
        rem_dst = jnp.where(lanes < 8, d - 8 + lanes, d - _L16 + lanes)

        def step(c, carry):
            for j in range(nfull):
                pltpu.async_copy(
                    table_hbm.at[idx_v.at[c], pl.ds(j * 128, 128)],
                    buf.at[:, pl.ds(j * 128, 128)], g0)
            pltpu.async_copy(tailt_hbm.at[idx_v.at[c]], tbuf, g0)
            for j in range(nfull):
                pltpu.make_async_copy(
                    table_hbm.at[idx_v.at[c], pl.ds(j * 128, 128)],
                    buf.at[:, pl.ds(j * 128, 128)], g0).wait()
            pltpu.make_async_copy(tailt_hbm.at[idx_v.at[c]], tbuf, g0).wait()

            def blit(r, carry2):
                for k in range(tail // _L16):
                    buf[r, pl.ds(nfull * 128 + k * _L16, _L16)] = (
                        tbuf[r, pl.ds(128 - tail + k * _L16, _L16)])
                if tail % _L16:
                    rfull = jnp.full((_L16,), r, jnp.int32)
                    vals = plsc.load_gather(tbuf, [rfull, rem_src])
                    plsc.store_scatter(buf, [rfull, rem_dst], vals)
                return carry2

            lax.fori_loop(0, l, blit, 0)

            pltpu.async_copy(buf, out_hbm.at[base + c], w0)
            pltpu.make_async_copy(buf, out_hbm.at[base + c], w0).wait()
            return carry

        lax.fori_loop(0, bpw, step, 0)

    return body


def kernel(idx, embed):
    b, l = idx.shape
    vocab, d = embed.shape
    assert b % (2 * _NW) == 0 and d > 128
    idx_r = idx.reshape(_NW, b // _NW, l).astype(jnp.int32)
    tail_table = embed[:, d - 128:]
    return _make_gather(b, l, vocab, d)(idx_r, embed, tail_table)


# speedup vs baseline: 1.7360x; 1.1240x over previous
import functools

import jax
import jax.numpy as jnp
from jax import lax
from jax.experimental import pallas as pl
from jax.experimental.pallas import tpu as pltpu
from jax.experimental.pallas import tpu_sc as plsc

_NC = 2
_NS = 16
_NW = _NC * _NS
_L16 = 16


def _make_gather(b: int, l: int, vocab: int, d: int):
    mesh = plsc.VectorSubcoreMesh(core_axis_name="c", subcore_axis_name="s")
    bpw = b // _NW
    nfull = d // 128
    tail = d - nfull * 128

    @functools.partial(
        pl.kernel,
        mesh=mesh,
        out_type=jax.ShapeDtypeStruct((b, l, d), jnp.float32),
        scratch_types=[
            pltpu.VMEM((bpw, l), jnp.int32),
            pltpu.VMEM((l, d), jnp.float32),
            pltpu.VMEM((l, d), jnp.float32),
            pltpu.VMEM((l, 128), jnp.float32),
            pltpu.SemaphoreType.DMA,
            pltpu.SemaphoreType.DMA,
            pltpu.SemaphoreType.DMA,
            pltpu.SemaphoreType.DMA,
            pltpu.SemaphoreType.DMA,
        ],
        compiler_params=pltpu.CompilerParams(needs_layout_passes=False),
    )
    def body(idx_hbm, table_hbm, tailt_hbm, out_hbm,
             idx_v, buf0, buf1, tbuf, g0, g1, ts, w0, w1):
        bufs = (buf0, buf1)
        gsem = (g0, g1)
        wsem = (w0, w1)
        wid = lax.axis_index("s") * _NC + lax.axis_index("c")
        base = wid * bpw
        pltpu.sync_copy(idx_hbm.at[wid], idx_v)

        lanes = lax.iota(jnp.int32, _L16)
        rem_src = jnp.where(lanes < 8, 128 - 8 + lanes, 128 - _L16 + lanes)
        rem_dst = jnp.where(lanes < 8, d - 8 + lanes, d - _L16 + lanes)

        def fire(c, buf):
            for j in range(nfull):
                pltpu.async_copy(
                    table_hbm.at[idx_v.at[c], pl.ds(j * 128, 128)],
                    buf.at[:, pl.ds(j * 128, 128)], gsem[buf is buf1])
            pltpu.async_copy(tailt_hbm.at[idx_v.at[c]], tbuf, ts)

        def make_blit(buf):
            def blit(r, carry2):
                for k in range(tail // _L16):
                    buf[r, pl.ds(nfull * 128 + k * _L16, _L16)] = (
                        tbuf[r, pl.ds(128 - tail + k * _L16, _L16)])
                if tail % _L16:
                    rfull = jnp.full((_L16,), r, jnp.int32)
                    vals = plsc.load_gather(tbuf, [rfull, rem_src])
                    plsc.store_scatter(buf, [rfull, rem_dst], vals)
                return carry2
            return blit

        fire(0, bufs[0])

        def pair(i, carry):
            for k in range(2):
                c = 2 * i + k
                nk = 1 - k
                pltpu.make_async_copy(tailt_hbm.at[idx_v.at[c]], tbuf, ts).wait()
                lax.fori_loop(0, l, make_blit(bufs[k]), 0)
                for j in range(nfull):
                    pltpu.make_async_copy(
                        table_hbm.at[idx_v.at[c], pl.ds(j * 128, 128)],
                        bufs[k].at[:, pl.ds(j * 128, 128)], gsem[k]).wait()

                @pl.when(c >= 1)
                def _():
                    pltpu.make_async_copy(
                        bufs[nk], out_hbm.at[base + c - 1], wsem[nk]).wait()

                @pl.when(c + 1 < bpw)
                def _():
                    fire(c + 1, bufs[nk])

                pltpu.async_copy(bufs[k], out_hbm.at[base + c], wsem[k])
            return carry

        lax.fori_loop(0, bpw // 2, pair, 0)

        c = bpw - 1
        pltpu.make_async_copy(
            bufs[c % 2], out_hbm.at[base + c], wsem[c % 2]).wait()

    return body


def kernel(idx, embed):
    b, l = idx.shape
    vocab, d = embed.shape
    assert b % (2 * _NW) == 0 and d > 128
    idx_r = idx.reshape(_NW, b // _NW, l).astype(jnp.int32)
    tail_table = embed[:, d - 128:]
    return _make_gather(b, l, vocab, d)(idx_r, embed, tail_table)


# speedup vs baseline: 1.7419x; 1.0034x over previous
import functools

import jax
import jax.numpy as jnp
from jax import lax
from jax.experimental import pallas as pl
from jax.experimental.pallas import tpu as pltpu
from jax.experimental.pallas import tpu_sc as plsc

_NC = 2
_NS = 16
_NW = _NC * _NS
_L16 = 16


def _make_gather(b: int, l: int, vocab: int, d: int):
    mesh = plsc.VectorSubcoreMesh(core_axis_name="c", subcore_axis_name="s")
    bpw = b // _NW
    nfull = d // 128
    tail = d - nfull * 128

    @functools.partial(
        pl.kernel,
        mesh=mesh,
        out_type=jax.ShapeDtypeStruct((b, l, d), jnp.float32),
        scratch_types=[
            pltpu.VMEM((bpw, l), jnp.int32),
            pltpu.VMEM((l, d), jnp.float32),
            pltpu.VMEM((l, d), jnp.float32),
            pltpu.VMEM((l, 128), jnp.float32),
            pltpu.SemaphoreType.DMA,
            pltpu.SemaphoreType.DMA,
            pltpu.SemaphoreType.DMA,
            pltpu.SemaphoreType.DMA,
            pltpu.SemaphoreType.DMA,
        ],
        compiler_params=pltpu.CompilerParams(needs_layout_passes=False),
    )
    def body(idx_hbm, table_hbm, tailt_hbm, out_hbm,
             idx_v, buf0, buf1, tbuf, g0, g1, ts, w0, w1):
        bufs = (buf0, buf1)
        gsem = (g0, g1)
        wsem = (w0, w1)
        wid = lax.axis_index("s") * _NC + lax.axis_index("c")
        base = wid * bpw
        pltpu.sync_copy(idx_hbm.at[wid], idx_v)

        lanes = lax.iota(jnp.int32, _L16)
        rem_src = jnp.where(lanes < 8, 128 - 8 + lanes, 128 - _L16 + lanes)
        rem_dst = jnp.where(lanes < 8, d - 8 + lanes, d - _L16 + lanes)

        def fire(c, buf):
            pltpu.async_copy(
                table_hbm.at[idx_v.at[c], pl.ds(0, nfull * 128)],
                buf.at[:, pl.ds(0, nfull * 128)], gsem[buf is buf1])
            pltpu.async_copy(tailt_hbm.at[idx_v.at[c]], tbuf, ts)

        def make_blit(buf):
            def blit(r, carry2):
                for k in range(tail // _L16):
                    buf[r, pl.ds(nfull * 128 + k * _L16, _L16)] = (
                        tbuf[r, pl.ds(128 - tail + k * _L16, _L16)])
                if tail % _L16:
                    rfull = jnp.full((_L16,), r, jnp.int32)
                    vals = plsc.load_gather(tbuf, [rfull, rem_src])
                    plsc.store_scatter(buf, [rfull, rem_dst], vals)
                return carry2
            return blit

        fire(0, bufs[0])

        def pair(i, carry):
            for k in range(2):
                c = 2 * i + k
                nk = 1 - k
                pltpu.make_async_copy(tailt_hbm.at[idx_v.at[c]], tbuf, ts).wait()
                lax.fori_loop(0, l, make_blit(bufs[k]), 0)
                pltpu.make_async_copy(
                    table_hbm.at[idx_v.at[c], pl.ds(0, nfull * 128)],
                    bufs[k].at[:, pl.ds(0, nfull * 128)], gsem[k]).wait()

                @pl.when(c >= 1)
                def _():
                    pltpu.make_async_copy(
                        bufs[nk], out_hbm.at[base + c - 1], wsem[nk]).wait()

                @pl.when(c + 1 < bpw)
                def _():
                    fire(c + 1, bufs[nk])

                pltpu.async_copy(bufs[k], out_hbm.at[base + c], wsem[k])
            return carry

        lax.fori_loop(0, bpw // 2, pair, 0)

        c = bpw - 1
        pltpu.make_async_copy(
            bufs[c % 2], out_hbm.at[base + c], wsem[c % 2]).wait()

    return body


def kernel(idx, embed):
    b, l = idx.shape
    vocab, d = embed.shape
    assert b % (2 * _NW) == 0 and d > 128
    idx_r = idx.reshape(_NW, b // _NW, l).astype(jnp.int32)
    tail_table = embed[:, d - 128:]
    return _make_gather(b, l, vocab, d)(idx_r, embed, tail_table)


# speedup vs baseline: 2.0732x; 1.1902x over previous
import functools

import jax
import jax.numpy as jnp
from jax import lax
from jax.experimental import pallas as pl
from jax.experimental.pallas import tpu as pltpu
from jax.experimental.pallas import tpu_sc as plsc

_NC = 2
_NS = 16
_NW = _NC * _NS
_L16 = 16


def _make_gather(b: int, l: int, vocab: int, d: int):
    mesh = plsc.VectorSubcoreMesh(core_axis_name="c", subcore_axis_name="s")
    bpw = b // _NW
    nfull = d // 128
    tail = d - nfull * 128

    @functools.partial(
        pl.kernel,
        mesh=mesh,
        out_type=jax.ShapeDtypeStruct((b, l, d), jnp.float32),
        scratch_types=[
            pltpu.VMEM((bpw, l), jnp.int32),
            pltpu.VMEM((l, d), jnp.float32),
            pltpu.VMEM((l, d), jnp.float32),
            pltpu.VMEM((l, 128), jnp.float32),
            pltpu.SemaphoreType.DMA,
            pltpu.SemaphoreType.DMA,
            pltpu.SemaphoreType.DMA,
            pltpu.SemaphoreType.DMA,
            pltpu.SemaphoreType.DMA,
        ],
        compiler_params=pltpu.CompilerParams(needs_layout_passes=False),
    )
    def body(idx_hbm, table_hbm, tailt_hbm, out_hbm,
             idx_v, buf0, buf1, tbuf, g0, g1, ts, w0, w1):
        bufs = (buf0, buf1)
        gsem = (g0, g1)
        wsem = (w0, w1)
        wid = lax.axis_index("s") * _NC + lax.axis_index("c")
        base = wid * bpw
        pltpu.sync_copy(idx_hbm.at[wid], idx_v)

        lanes = lax.iota(jnp.int32, _L16)
        rem_src = jnp.where(lanes < 8, 128 - 8 + lanes, 128 - _L16 + lanes)
        rem_dst = jnp.where(lanes < 8, d - 8 + lanes, d - _L16 + lanes)

        def fire(c, buf):
            for j in range(nfull):
                pltpu.async_copy(
                    table_hbm.at[idx_v.at[c], pl.ds(j * 128, 128)],
                    buf.at[:, pl.ds(j * 128, 128)], gsem[buf is buf1])
            pltpu.async_copy(tailt_hbm.at[idx_v.at[c]], tbuf, ts)

        def make_blit(buf):
            def blit(r, carry2):
                for k in range(tail // _L16):
                    buf[r, pl.ds(nfull * 128 + k * _L16, _L16)] = (
                        tbuf[r, pl.ds(128 - tail + k * _L16, _L16)])
                if tail % _L16:
                    rfull = jnp.full((_L16,), r, jnp.int32)
                    vals = plsc.load_gather(tbuf, [rfull, rem_src])
                    plsc.store_scatter(buf, [rfull, rem_dst], vals)
                return carry2
            return blit

        fire(0, bufs[0])

        def pair(i, carry):
            for k in range(2):
                c = 2 * i + k
                nk = 1 - k
                pltpu.make_async_copy(tailt_hbm.at[idx_v.at[c]], tbuf, ts).wait()
                lax.fori_loop(0, l, make_blit(bufs[k]), 0)
                for j in range(nfull):
                    pltpu.make_async_copy(
                        table_hbm.at[idx_v.at[c], pl.ds(j * 128, 128)],
                        bufs[k].at[:, pl.ds(j * 128, 128)], gsem[k]).wait()

                @pl.when(c >= 1)
                def _():
                    pltpu.make_async_copy(
                        bufs[nk], out_hbm.at[base + c - 1], wsem[nk]).wait()

                @pl.when(c + 1 < bpw)
                def _():
                    fire(c + 1, bufs[nk])

                pltpu.async_copy(bufs[k], out_hbm.at[base + c], wsem[k])
            return carry

        lax.fori_loop(0, bpw // 2, pair, 0)

        c = bpw - 1
        pltpu.make_async_copy(
            bufs[c % 2], out_hbm.at[base + c], wsem[c % 2]).wait()

    return body


def kernel(idx, embed):
    b, l = idx.shape
    vocab, d = embed.shape
    assert b % (2 * _NW) == 0 and d > 128
    idx_r = idx.reshape(_NW, b // _NW, l).astype(jnp.int32)
    tail_table = embed[:, d - 128:]
    out = _make_gather(b, l, vocab, d)(idx_r, embed, tail_table)
    return lax.optimization_barrier(out)
